# baseline (speedup 1.0000x reference)
import functools

import jax
import jax.numpy as jnp
from jax import lax
from jax.experimental import pallas as pl
from jax.experimental.pallas import tpu as pltpu
from jax.experimental.pallas import tpu_sc as plsc

NW = 32
LANE = 128
SUB = 8
B_TILES_PER_W = 4
CHUNK_T = 2
CHUNK_B = CHUNK_T * LANE
TRANSPOSE = True


@functools.partial(jax.jit, static_argnums=(2, 3, 4))
def _embedding_gather(idx3, table, hist, bsz, d):
    jr_t = d // SUB
    b_tiles = bsz // LANE
    n_chunks = hist * (B_TILES_PER_W // CHUNK_T)
    mesh = plsc.VectorSubcoreMesh(core_axis_name="c", subcore_axis_name="s")

    @functools.partial(
        pl.kernel,
        mesh=mesh,
        out_type=jax.ShapeDtypeStruct((hist, jr_t, b_tiles, SUB, LANE), jnp.float32),
        compiler_params=pltpu.CompilerParams(
            use_tc_tiling_on_sc=False, needs_layout_passes=False
        ),
        scratch_types=[
            pltpu.VMEM((2, CHUNK_T, LANE), jnp.int32),
            pltpu.VMEM((2, CHUNK_B, d), jnp.float32),
            pltpu.VMEM((2, jr_t, CHUNK_T, SUB, LANE), jnp.float32),
            pltpu.SemaphoreType.DMA,
            pltpu.SemaphoreType.DMA,
            pltpu.SemaphoreType.DMA,
        ],
    )
    def body(idx_hbm, tab_hbm, out_hbm, idx_v, rows_v, rowsT_v, sem_i, sem_g, sem_o):
        wid = lax.axis_index("s") * 2 + lax.axis_index("c")
        ctile0 = wid * B_TILES_PER_W
        lane_iota = lax.iota(jnp.int32, 16)

        def h_of(t):
            return t // 2

        def cbase_of(t):
            return ctile0 + (t % 2) * CHUNK_T

        def idx_fetch(t, s):
            pltpu.async_copy(
                idx_hbm.at[h_of(t), pl.ds(cbase_of(t), CHUNK_T)], idx_v.at[s], sem_i
            )

        def idx_drain(s):
            pltpu.make_async_copy(
                idx_hbm.at[0, pl.ds(ctile0, CHUNK_T)], idx_v.at[s], sem_i
            ).wait()

        def gathers_issue(s):
            for cc in range(CHUNK_T):
                pltpu.async_copy(
                    tab_hbm.at[idx_v.at[s, cc]],
                    rows_v.at[s, pl.ds(cc * LANE, LANE)],
                    sem_g,
                )

        def gathers_drain(s):
            for cc in range(CHUNK_T):
                pltpu.make_async_copy(
                    tab_hbm.at[idx_v.at[s, cc]],
                    rows_v.at[s, pl.ds(cc * LANE, LANE)],
                    sem_g,
                ).wait()

        def wb_issue(t, s):
            pltpu.async_copy(
                rowsT_v.at[s], out_hbm.at[h_of(t), :, pl.ds(cbase_of(t), CHUNK_T)], sem_o
            )

        def wb_drain(t, s):
            pltpu.make_async_copy(
                rowsT_v.at[s], out_hbm.at[h_of(t), :, pl.ds(cbase_of(t), CHUNK_T)], sem_o
            ).wait()

        def transpose(s):
            @plsc.parallel_loop(0, CHUNK_B // 16, 1, unroll=2)
            def bgroup(gi):
                b_idx = gi * 16 + lane_iota
                cp = gi // 8
                off = (gi % 8) * 16
                def load16(j0):
                    return [
                        plsc.load_gather(
                            rows_v.at[s],
                            [b_idx, jnp.full((16,), j0 + k, jnp.int32)],
                        )
                        for k in range(16)
                    ]

                def store16(j0, vs):
                    for k in range(16):
                        j = j0 + k
                        rowsT_v[s, j // SUB, cp, j % SUB, pl.ds(off, 16)] = vs[k]

                prev = load16(0)
                for j0 in (16, 32, 48):
                    cur = load16(j0)
                    store16(j0 - 16, prev)
                    prev = cur
                store16(48, prev)

        idx_fetch(0, 0)
        idx_fetch(1, 1)
        idx_drain(0)
        gathers_issue(0)

        def step(t, carry):
            s = t % 2
            sn = (t + 1) % 2
            gathers_drain(s)
            @pl.when(t + 2 < n_chunks)
            def _():
                idx_fetch(t + 2, s)

            @pl.when(t + 1 < n_chunks)
            def _():
                idx_drain(sn)
                gathers_issue(sn)

            @pl.when(t >= 2)
            def _():
                wb_drain(t, s)

            if TRANSPOSE:
                transpose(s)
            wb_issue(t, s)
            return carry

        lax.fori_loop(0, n_chunks, step, 0)
        wb_drain(n_chunks - 2, 0)
        wb_drain(n_chunks - 1, 1)

    return body(idx3, table)


def kernel(x, embed_matrix):
    bsz, hist = x.shape
    v, d = embed_matrix.shape
    idx3 = jnp.transpose(x).reshape(hist, bsz // LANE, LANE).astype(jnp.int32)
    out5 = _embedding_gather(idx3, embed_matrix, hist, bsz, d)
    return out5.transpose(2, 4, 0, 1, 3).reshape(bsz, hist, d)


# speedup vs baseline: 1.9499x; 1.9499x over previous
import functools

import jax
import jax.numpy as jnp
from jax import lax
from jax.experimental import pallas as pl
from jax.experimental.pallas import tpu as pltpu
from jax.experimental.pallas import tpu_sc as plsc

NW = 32
LANE = 128
SUB = 8
B_TILES_PER_W = 4
CHUNK_T = 2
CHUNK_B = CHUNK_T * LANE
PITCH = LANE + 1


@functools.partial(jax.jit, static_argnums=(2, 3, 4))
def _embedding_gather(idx3, table, hist, bsz, d):
    jr_t = d // SUB
    b_tiles = bsz // LANE
    n_chunks = hist * (B_TILES_PER_W // CHUNK_T)
    mesh = plsc.VectorSubcoreMesh(core_axis_name="c", subcore_axis_name="s")

    @functools.partial(
        pl.kernel,
        mesh=mesh,
        out_type=jax.ShapeDtypeStruct((hist, jr_t, b_tiles, SUB, LANE), jnp.float32),
        compiler_params=pltpu.CompilerParams(
            use_tc_tiling_on_sc=False, needs_layout_passes=False
        ),
        scratch_types=[
            pltpu.VMEM((2, CHUNK_T, LANE), jnp.int32),
            pltpu.VMEM((2, CHUNK_B, d), jnp.float32),
            pltpu.VMEM((2, CHUNK_T, d, PITCH), jnp.float32),
            pltpu.SemaphoreType.DMA,
            pltpu.SemaphoreType.DMA,
            pltpu.SemaphoreType.DMA,
        ],
    )
    def body(idx_hbm, tab_hbm, out_hbm, idx_v, rows_v, rowsT_v, sem_i, sem_g, sem_o):
        wid = lax.axis_index("s") * 2 + lax.axis_index("c")
        ctile0 = wid * B_TILES_PER_W
        lane_iota = lax.iota(jnp.int32, 16)

        def h_of(t):
            return t // 2

        def cbase_of(t):
            return ctile0 + (t % 2) * CHUNK_T

        def idx_fetch(t, s):
            pltpu.async_copy(
                idx_hbm.at[h_of(t), pl.ds(cbase_of(t), CHUNK_T)], idx_v.at[s], sem_i
            )

        def idx_drain(s):
            pltpu.make_async_copy(
                idx_hbm.at[0, pl.ds(ctile0, CHUNK_T)], idx_v.at[s], sem_i
            ).wait()

        def gathers_issue(s):
            for cc in range(CHUNK_T):
                pltpu.async_copy(
                    tab_hbm.at[idx_v.at[s, cc]],
                    rows_v.at[s, pl.ds(cc * LANE, LANE)],
                    sem_g,
                )

        def gathers_drain(s):
            for cc in range(CHUNK_T):
                pltpu.make_async_copy(
                    tab_hbm.at[idx_v.at[s, cc]],
                    rows_v.at[s, pl.ds(cc * LANE, LANE)],
                    sem_g,
                ).wait()

        def wb_issue(t, s):
            for cp in range(CHUNK_T):
                for jt in range(jr_t):
                    pltpu.async_copy(
                        rowsT_v.at[s, cp, pl.ds(jt * SUB, SUB), pl.ds(0, LANE)],
                        out_hbm.at[h_of(t), jt, cbase_of(t) + cp],
                        sem_o,
                    )

        def wb_drain(t, s):
            for cp in range(CHUNK_T):
                for jt in range(jr_t):
                    pltpu.make_async_copy(
                        rowsT_v.at[s, cp, pl.ds(jt * SUB, SUB), pl.ds(0, LANE)],
                        out_hbm.at[h_of(t), jt, cbase_of(t) + cp],
                        sem_o,
                    ).wait()

        def transpose(s):
            for cp in range(CHUNK_T):
                rT = rowsT_v.at[s, cp]
                jrows = [j0 * 16 + lane_iota for j0 in range(d // 16)]

                @plsc.parallel_loop(0, LANE, 1, unroll=2)
                def brow(b):
                    bvec = jnp.full((16,), b, jnp.int32)
                    for j0 in range(d // 16):
                        v = rows_v[s, cp * LANE + b, pl.ds(j0 * 16, 16)]
                        plsc.store_scatter(rT, [jrows[j0], bvec], v)

        idx_fetch(0, 0)
        idx_fetch(1, 1)
        idx_drain(0)
        gathers_issue(0)

        def step(t, carry):
            s = t % 2
            sn = (t + 1) % 2
            gathers_drain(s)
            @pl.when(t + 2 < n_chunks)
            def _():
                idx_fetch(t + 2, s)

            @pl.when(t + 1 < n_chunks)
            def _():
                idx_drain(sn)
                gathers_issue(sn)

            @pl.when(t >= 2)
            def _():
                wb_drain(t, s)

            transpose(s)
            wb_issue(t, s)
            return carry

        lax.fori_loop(0, n_chunks, step, 0)
        wb_drain(n_chunks - 2, 0)
        wb_drain(n_chunks - 1, 1)

    return body(idx3, table)


def kernel(x, embed_matrix):
    bsz, hist = x.shape
    v, d = embed_matrix.shape
    idx3 = jnp.transpose(x).reshape(hist, bsz // LANE, LANE).astype(jnp.int32)
    out5 = _embedding_gather(idx3, embed_matrix, hist, bsz, d)
    return out5.transpose(2, 4, 0, 1, 3).reshape(bsz, hist, d)
